# Initial kernel scaffold; baseline (speedup 1.0000x reference)
import jax
import jax.numpy as jnp
from jax.experimental import pallas as pl


def kernel(x, pos_table):
    raise NotImplementedError("write your pallas kernel here")


# speedup vs baseline: 2.1889x; 2.1889x over previous
import jax
import jax.numpy as jnp
from jax.experimental import pallas as pl

_BLOCK_S = 256


def _add_kernel(x_ref, pos_ref, o_ref):
    o_ref[...] = x_ref[...] + pos_ref[...]


def kernel(x, pos_table):
    B, S, D = x.shape
    pos = pos_table[:S]
    grid = (S // _BLOCK_S, B)
    return pl.pallas_call(
        _add_kernel,
        grid=grid,
        in_specs=[
            pl.BlockSpec((1, _BLOCK_S, D), lambda s, b: (b, s, 0)),
            pl.BlockSpec((_BLOCK_S, D), lambda s, b: (s, 0)),
        ],
        out_specs=pl.BlockSpec((1, _BLOCK_S, D), lambda s, b: (b, s, 0)),
        out_shape=jax.ShapeDtypeStruct(x.shape, x.dtype),
    )(x, pos)


# speedup vs baseline: 3.2071x; 1.4652x over previous
import jax
import jax.numpy as jnp
from jax.experimental import pallas as pl

_BLOCK_S = 256


def _add_kernel(x_ref, pos_ref, o_ref):
    o_ref[...] = x_ref[...] + pos_ref[...][None]


def kernel(x, pos_table):
    B, S, D = x.shape
    pos = pos_table[:S]
    grid = (S // _BLOCK_S,)
    return pl.pallas_call(
        _add_kernel,
        grid=grid,
        in_specs=[
            pl.BlockSpec((B, _BLOCK_S, D), lambda s: (0, s, 0)),
            pl.BlockSpec((_BLOCK_S, D), lambda s: (s, 0)),
        ],
        out_specs=pl.BlockSpec((B, _BLOCK_S, D), lambda s: (0, s, 0)),
        out_shape=jax.ShapeDtypeStruct(x.shape, x.dtype),
    )(x, pos)


# speedup vs baseline: 3.2101x; 1.0009x over previous
import jax
import jax.numpy as jnp
from jax.experimental import pallas as pl

_BLOCK_S = 512


def _add_kernel(x_ref, pos_ref, o_ref):
    o_ref[...] = x_ref[...] + pos_ref[...][None]


def kernel(x, pos_table):
    B, S, D = x.shape
    pos = pos_table[:S]
    grid = (S // _BLOCK_S,)
    return pl.pallas_call(
        _add_kernel,
        grid=grid,
        in_specs=[
            pl.BlockSpec((B, _BLOCK_S, D), lambda s: (0, s, 0)),
            pl.BlockSpec((_BLOCK_S, D), lambda s: (s, 0)),
        ],
        out_specs=pl.BlockSpec((B, _BLOCK_S, D), lambda s: (0, s, 0)),
        out_shape=jax.ShapeDtypeStruct(x.shape, x.dtype),
    )(x, pos)
